# Initial kernel scaffold; baseline (speedup 1.0000x reference)
import jax
import jax.numpy as jnp
from jax.experimental import pallas as pl


def kernel(lang_ids, embeddings):
    raise NotImplementedError("write your pallas kernel here")


# speedup vs baseline: 1.0832x; 1.0832x over previous
import functools

import jax
import jax.numpy as jnp
from jax import lax
from jax.experimental import pallas as pl
from jax.experimental.pallas import tpu as pltpu
from jax.experimental.pallas import tpu_sc as plsc

VOCAB = 101
D_MODEL = 1024
B_TOTAL = 4 * 4096

_INFO = plsc.get_sparse_core_info()
_NC, _NS = _INFO.num_cores, _INFO.num_subcores
_NW = _NC * _NS
_BPW = B_TOTAL // _NW
_CHUNK = 64
_NCHUNK = _BPW // _CHUNK


def _body(table_hbm, ids_hbm, out_hbm, idx_v, rows_v, gsem, wsem):
    wid = lax.axis_index("s") * _NC + lax.axis_index("c")
    base = wid * _BPW
    pltpu.sync_copy(ids_hbm.at[pl.ds(base, _BPW)], idx_v)
    for c in range(_NCHUNK):
        idx_slice = idx_v.at[pl.ds(c * _CHUNK, _CHUNK)]
        pltpu.async_copy(table_hbm.at[idx_slice], rows_v, gsem).wait()
        pltpu.async_copy(
            rows_v, out_hbm.at[pl.ds(base + c * _CHUNK, _CHUNK)], wsem
        ).wait()


@jax.jit
def _run(ids_flat, embeddings):
    mesh = plsc.VectorSubcoreMesh(core_axis_name="c", subcore_axis_name="s")
    k = pl.kernel(
        _body,
        out_type=jax.ShapeDtypeStruct((B_TOTAL, D_MODEL), jnp.float32),
        mesh=mesh,
        scratch_types=[
            pltpu.VMEM((_BPW,), jnp.int32),
            pltpu.VMEM((_CHUNK, D_MODEL), jnp.float32),
            pltpu.SemaphoreType.DMA,
            pltpu.SemaphoreType.DMA,
        ],
    )
    return k(embeddings, ids_flat)


def kernel(lang_ids, embeddings):
    ids_flat = lang_ids.reshape(-1).astype(jnp.int32)
    out = _run(ids_flat, embeddings)
    return out.reshape(lang_ids.shape + (D_MODEL,))


# speedup vs baseline: 1.0838x; 1.0005x over previous
import functools

import jax
import jax.numpy as jnp
from jax import lax
from jax.experimental import pallas as pl
from jax.experimental.pallas import tpu as pltpu
from jax.experimental.pallas import tpu_sc as plsc

VOCAB = 101
D_MODEL = 1024
B_TOTAL = 4 * 4096

_INFO = plsc.get_sparse_core_info()
_NC, _NS = _INFO.num_cores, _INFO.num_subcores
_NW = _NC * _NS
_BPW = B_TOTAL // _NW
_CHUNK = 32
_NCHUNK = _BPW // _CHUNK
_NBUF = 3


def _body(table_hbm, ids_hbm, out_hbm, idx_v, rows0, rows1, rows2,
          gsem0, gsem1, gsem2, wsem0, wsem1, wsem2):
    wid = lax.axis_index("s") * _NC + lax.axis_index("c")
    base = wid * _BPW
    pltpu.sync_copy(ids_hbm.at[pl.ds(base, _BPW)], idx_v)
    rows = (rows0, rows1, rows2)
    gsem = (gsem0, gsem1, gsem2)
    wsem = (wsem0, wsem1, wsem2)

    def gather_copy(c):
        b = c % _NBUF
        idx_slice = idx_v.at[pl.ds(c * _CHUNK, _CHUNK)]
        return pltpu.make_async_copy(table_hbm.at[idx_slice], rows[b], gsem[b])

    def write_copy(c):
        b = c % _NBUF
        return pltpu.make_async_copy(
            rows[b], out_hbm.at[pl.ds(base + c * _CHUNK, _CHUNK)], wsem[b])

    for i in range(_NBUF):
        gather_copy(i).start()
    for c in range(_NCHUNK):
        gather_copy(c).wait()
        write_copy(c).start()
        prev = c - (_NBUF - 1)
        if prev >= 0 and prev + _NBUF < _NCHUNK:
            write_copy(prev).wait()
            gather_copy(prev + _NBUF).start()
    for c in range(_NCHUNK - _NBUF, _NCHUNK):
        write_copy(c).wait()


@jax.jit
def _run(ids_flat, embeddings):
    mesh = plsc.VectorSubcoreMesh(core_axis_name="c", subcore_axis_name="s")
    k = pl.kernel(
        _body,
        out_type=jax.ShapeDtypeStruct((B_TOTAL, D_MODEL), jnp.float32),
        mesh=mesh,
        scratch_types=[
            pltpu.VMEM((_BPW,), jnp.int32),
            pltpu.VMEM((_CHUNK, D_MODEL), jnp.float32),
            pltpu.VMEM((_CHUNK, D_MODEL), jnp.float32),
            pltpu.VMEM((_CHUNK, D_MODEL), jnp.float32),
            pltpu.SemaphoreType.DMA,
            pltpu.SemaphoreType.DMA,
            pltpu.SemaphoreType.DMA,
            pltpu.SemaphoreType.DMA,
            pltpu.SemaphoreType.DMA,
            pltpu.SemaphoreType.DMA,
        ],
    )
    return k(embeddings, ids_flat)


def kernel(lang_ids, embeddings):
    ids_flat = lang_ids.reshape(-1).astype(jnp.int32)
    out = _run(ids_flat, embeddings)
    return out.reshape(lang_ids.shape + (D_MODEL,))
